# baseline (speedup 1.0000x reference)
import functools

import jax
import jax.numpy as jnp
from jax import lax
from jax._src import core as _jax_core
from jax._src.pallas import core as _pallas_core
from jax.experimental import pallas as pl
from jax.experimental.pallas import tpu as pltpu
from jax.experimental.pallas import tpu_sc as plsc

VOCAB = 1000000
EMBED = 64
BATCH = 16384

_HALF = 256
_NSEM = 8


def _to_default_space(x):
  return _pallas_core.with_memory_space_constraint_p.bind(
      x, memory_space=_jax_core.MemorySpace.Device)


def _build_kernel():
  info = plsc.get_sparse_core_info()
  nc, ns = info.num_cores, info.num_subcores
  nw = nc * ns
  b_per_w = BATCH // nw
  n_halves = b_per_w // _HALF

  mesh = plsc.VectorSubcoreMesh(core_axis_name="c", subcore_axis_name="s")

  @functools.partial(
      pl.kernel,
      mesh=mesh,
      out_type=(
          pltpu.HBM((BATCH, EMBED), jnp.float32),
          pltpu.HBM((BATCH, EMBED), jnp.float32),
      ),
      scratch_types=[
          pltpu.VMEM((b_per_w,), jnp.int32),
          pltpu.VMEM((b_per_w,), jnp.int32),
          pltpu.VMEM((_HALF, EMBED), jnp.float32),
          pltpu.VMEM((_HALF, EMBED), jnp.float32),
          [pltpu.SemaphoreType.DMA] * _NSEM,
          [pltpu.SemaphoreType.DMA] * _NSEM,
      ],
  )
  def lookup(center_hbm, context_hbm, word_hbm, ctx_hbm,
             out_c, out_x, cidx_v, xidx_v, crows_v, xrows_v, csems, xsems):
    wid = lax.axis_index("s") * nc + lax.axis_index("c")
    base = wid * b_per_w

    pltpu.sync_copy(center_hbm.at[pl.ds(base, b_per_w)], cidx_v)
    pltpu.sync_copy(context_hbm.at[pl.ds(base, b_per_w)], xidx_v)

    for half in range(n_halves):
      def group_body(g, _):
        cv = cidx_v[pl.ds(half * _HALF + g * 16, 16)]
        xv = xidx_v[pl.ds(half * _HALF + g * 16, 16)]
        for lane in range(16):
          pltpu.async_copy(word_hbm.at[pl.ds(cv[lane], 1)],
                           crows_v.at[pl.ds(g * 16 + lane, 1)],
                           csems[lane % _NSEM])
          pltpu.async_copy(ctx_hbm.at[pl.ds(xv[lane], 1)],
                           xrows_v.at[pl.ds(g * 16 + lane, 1)],
                           xsems[lane % _NSEM])
        return 0

      lax.fori_loop(0, _HALF // 16, group_body, 0)

      for s in range(_NSEM):
        pltpu.make_async_copy(
            word_hbm.at[pl.ds(0, _HALF // _NSEM)],
            crows_v.at[pl.ds(0, _HALF // _NSEM)], csems[s]).wait()
        pltpu.make_async_copy(
            ctx_hbm.at[pl.ds(0, _HALF // _NSEM)],
            xrows_v.at[pl.ds(0, _HALF // _NSEM)], xsems[s]).wait()

      pltpu.sync_copy(crows_v, out_c.at[pl.ds(base + half * _HALF, _HALF)])
      pltpu.sync_copy(xrows_v, out_x.at[pl.ds(base + half * _HALF, _HALF)])

  return lookup


_lookup = _build_kernel()


@jax.jit
def kernel(center, context, word_embeds, context_embeds):
  out_c, out_x = _lookup(center.astype(jnp.int32), context.astype(jnp.int32),
                         word_embeds, context_embeds)
  return _to_default_space(out_c), _to_default_space(out_x)
